# Initial kernel scaffold; baseline (speedup 1.0000x reference)
import jax
import jax.numpy as jnp
from jax.experimental import pallas as pl


def kernel(x, weights, in_idxs):
    raise NotImplementedError("write your pallas kernel here")


# speedup vs baseline: 56.5159x; 56.5159x over previous
import functools

import jax
import jax.numpy as jnp
from jax import lax
from jax.experimental import pallas as pl
from jax.experimental.pallas import tpu as pltpu
from jax.experimental.pallas import tpu_sc as plsc

_IN_SIZE = 1024
_N_NODES = 2048
_FAN_IN = 128
_OUT_SIZE = 16
_BATCH = 4096
_R_TOTAL = _IN_SIZE + _N_NODES

_N_WORKERS = 32
_NODES_PER_WORKER = _N_NODES // _N_WORKERS


def _build_mt_kernel(w_hbm, idx_hbm, mt_hbm, d8_hbm, w_v, idx_v, row_v, d8_v):
    wid = lax.axis_index("s") * 2 + lax.axis_index("c")
    base = wid * _NODES_PER_WORKER
    pltpu.sync_copy(w_hbm.at[pl.ds(base, _NODES_PER_WORKER)], w_v)
    pltpu.sync_copy(idx_hbm.at[pl.ds(base, _NODES_PER_WORKER)], idx_v)

    zeros16 = jnp.zeros((16,), jnp.float32)

    def zero_body(i, carry):
        row_v[pl.ds(i * 16, 16)] = zeros16
        return carry

    lax.fori_loop(0, _R_TOTAL // 16, zero_body, 0)
    d8_v[...] = zeros16

    def node_body(n, carry):
        node = base + n
        limit = _IN_SIZE + node
        group_ref_base = limit - lax.rem(node, 8)
        for v in range(_FAN_IN // 16):
            iv = idx_v[n, pl.ds(v * 16, 16)]
            wv = w_v[n, pl.ds(v * 16, 16)]
            iw = lax.bitcast_convert_type(wv, jnp.int32)
            lsb = jnp.bitwise_and(lax.shift_right_logical(iw, 16), 1)
            ir = jnp.bitwise_and(iw + 0x7FFF + lsb, jnp.int32(-65536))
            wr = lax.bitcast_convert_type(ir, jnp.float32)
            keep = iv < limit
            wv = jnp.where(keep, wr, 0.0)
            plsc.addupdate_scatter(row_v, [iv], wv)
            iv2 = iv - group_ref_base
            ing = jnp.logical_and(iv2 >= 0, keep)
            wv2 = jnp.where(ing, wr, 0.0)
            iv2c = jnp.clip(iv2, 0, 15)
            plsc.addupdate_scatter(d8_v, [iv2c], wv2)
        pltpu.sync_copy(row_v, mt_hbm.at[node])
        pltpu.sync_copy(d8_v, d8_hbm.at[node])
        for v in range(_FAN_IN // 16):
            iv = idx_v[n, pl.ds(v * 16, 16)]
            plsc.store_scatter(row_v, [iv], zeros16)
        d8_v[...] = zeros16
        return carry

    lax.fori_loop(0, _NODES_PER_WORKER, node_body, 0)


@functools.cache
def _build_mt():
    return pl.kernel(
        _build_mt_kernel,
        out_type=(
            jax.ShapeDtypeStruct((_N_NODES, _R_TOTAL), jnp.float32),
            jax.ShapeDtypeStruct((_N_NODES, 16), jnp.float32),
        ),
        mesh=plsc.VectorSubcoreMesh(core_axis_name="c", subcore_axis_name="s"),
        scratch_types=[
            pltpu.VMEM((_NODES_PER_WORKER, _FAN_IN), jnp.float32),
            pltpu.VMEM((_NODES_PER_WORKER, _FAN_IN), jnp.int32),
            pltpu.VMEM((_R_TOTAL,), jnp.float32),
            pltpu.VMEM((16,), jnp.float32),
        ],
        compiler_params=pltpu.CompilerParams(needs_layout_passes=False),
    )


def _recur_body(blk, b_tile, mt_ref, d8_ref, xT_ref, out_ref, acc_s, out_s, xf_s):
    n_blks = _N_NODES // blk
    xf_s[...] = xT_ref[...].astype(jnp.float32)
    for b in range(n_blks):
        bs = b * blk
        acc = jnp.dot(
            mt_ref[bs : bs + blk, : _IN_SIZE],
            xf_s[...],
            preferred_element_type=jnp.float32,
            precision=lax.Precision.HIGHEST,
        )
        if b > 0:
            acc = acc + jnp.dot(
                mt_ref[bs : bs + blk, _IN_SIZE : _IN_SIZE + bs],
                out_s[0:bs, :],
                preferred_element_type=jnp.float32,
                precision=lax.Precision.HIGHEST,
            )
        acc_s[...] = acc
        out_s[bs : bs + blk, :] = jnp.zeros((blk, b_tile), jnp.float32)

        def do_group(g, out_slot=None):
            off8 = pl.multiple_of(g * 8, 8)
            base8 = pl.multiple_of(bs + g * 8, 8)
            m8 = mt_ref[pl.ds(base8, 8), _IN_SIZE + bs : _IN_SIZE + bs + blk]
            acc8 = acc_s[pl.ds(off8, 8), :] + jnp.dot(
                m8,
                out_s[bs : bs + blk, :],
                preferred_element_type=jnp.float32,
                precision=lax.Precision.HIGHEST,
            )
            d8 = d8_ref[pl.ds(base8, 8), :]
            rows_r, rows_f = [], []
            for k in range(8):
                v = acc8[k : k + 1, :]
                for j in range(k):
                    v = v + d8[k : k + 1, j : j + 1] * rows_r[j]
                t = jnp.tanh(v)
                rows_f.append(t)
                rows_r.append(t.astype(jnp.bfloat16).astype(jnp.float32))
            out_s[pl.ds(base8, 8), :] = jnp.concatenate(rows_r, axis=0)
            if out_slot is not None:
                out_ref[out_slot * 8 : (out_slot + 1) * 8, :] = jnp.concatenate(
                    rows_f, axis=0
                )

        n_groups = blk // 8
        if b < n_blks - 1:
            lax.fori_loop(0, n_groups, lambda g, c: (do_group(g), c)[1], 0)
        else:
            lax.fori_loop(0, n_groups - 2, lambda g, c: (do_group(g), c)[1], 0)
            do_group(n_groups - 2, out_slot=0)
            do_group(n_groups - 1, out_slot=1)


def _make_recurrence(batch, b_tile, blk, interpret=False):
    return pl.pallas_call(
        functools.partial(_recur_body, blk, b_tile),
        grid=(batch // b_tile,),
        in_specs=[
            pl.BlockSpec((_N_NODES, _R_TOTAL), lambda i: (0, 0)),
            pl.BlockSpec((_N_NODES, 16), lambda i: (0, 0)),
            pl.BlockSpec((_IN_SIZE, b_tile), lambda i: (0, i)),
        ],
        out_specs=pl.BlockSpec((_OUT_SIZE, b_tile), lambda i: (0, i)),
        out_shape=jax.ShapeDtypeStruct((_OUT_SIZE, batch), jnp.float32),
        scratch_shapes=[
            pltpu.VMEM((blk, b_tile), jnp.float32),
            pltpu.VMEM((_N_NODES, b_tile), jnp.float32),
            pltpu.VMEM((_IN_SIZE, b_tile), jnp.float32),
        ],
        interpret=interpret,
    )


def kernel(x, weights, in_idxs):
    mt, d8 = _build_mt()(weights, in_idxs.astype(jnp.int32))
    xT = x.T.astype(jnp.bfloat16)
    return _make_recurrence(_BATCH, 1024, 128)(mt, d8, xT)


# speedup vs baseline: 61.5433x; 1.0890x over previous
import functools

import jax
import jax.numpy as jnp
from jax import lax
from jax.experimental import pallas as pl
from jax.experimental.pallas import tpu as pltpu
from jax.experimental.pallas import tpu_sc as plsc

_IN_SIZE = 1024
_N_NODES = 2048
_FAN_IN = 128
_OUT_SIZE = 16
_BATCH = 4096
_R_TOTAL = _IN_SIZE + _N_NODES

_N_WORKERS = 32
_NODES_PER_WORKER = _N_NODES // _N_WORKERS


def _build_mt_kernel(w_hbm, idx_hbm, mt_hbm, d8_hbm, w_v, idx_v, row_v, d8_v):
    wid = lax.axis_index("s") * 2 + lax.axis_index("c")
    base = wid * _NODES_PER_WORKER
    pltpu.sync_copy(w_hbm.at[pl.ds(base, _NODES_PER_WORKER)], w_v)
    pltpu.sync_copy(idx_hbm.at[pl.ds(base, _NODES_PER_WORKER)], idx_v)

    zeros16 = jnp.zeros((16,), jnp.float32)

    def zero_body(i, carry):
        row_v[pl.ds(i * 16, 16)] = zeros16
        return carry

    lax.fori_loop(0, _R_TOTAL // 16, zero_body, 0)
    d8_v[...] = zeros16

    def node_body(n, carry):
        node = base + n
        limit = _IN_SIZE + node
        group_ref_base = limit - lax.rem(node, 16)
        for v in range(_FAN_IN // 16):
            iv = idx_v[n, pl.ds(v * 16, 16)]
            wv = w_v[n, pl.ds(v * 16, 16)]
            iw = lax.bitcast_convert_type(wv, jnp.int32)
            lsb = jnp.bitwise_and(lax.shift_right_logical(iw, 16), 1)
            ir = jnp.bitwise_and(iw + 0x7FFF + lsb, jnp.int32(-65536))
            wr = lax.bitcast_convert_type(ir, jnp.float32)
            keep = iv < limit
            wv = jnp.where(keep, wr, 0.0)
            plsc.addupdate_scatter(row_v, [iv], wv)
            iv2 = iv - group_ref_base
            ing = jnp.logical_and(iv2 >= 0, keep)
            wv2 = jnp.where(ing, wr, 0.0)
            iv2c = jnp.clip(iv2, 0, 15)
            plsc.addupdate_scatter(d8_v, [iv2c], wv2)
        pltpu.sync_copy(row_v, mt_hbm.at[node])
        pltpu.sync_copy(d8_v, d8_hbm.at[node])
        for v in range(_FAN_IN // 16):
            iv = idx_v[n, pl.ds(v * 16, 16)]
            plsc.store_scatter(row_v, [iv], zeros16)
        d8_v[...] = zeros16
        return carry

    lax.fori_loop(0, _NODES_PER_WORKER, node_body, 0)


@functools.cache
def _build_mt():
    return pl.kernel(
        _build_mt_kernel,
        out_type=(
            jax.ShapeDtypeStruct((_N_NODES, _R_TOTAL), jnp.float32),
            jax.ShapeDtypeStruct((_N_NODES, 16), jnp.float32),
        ),
        mesh=plsc.VectorSubcoreMesh(core_axis_name="c", subcore_axis_name="s"),
        scratch_types=[
            pltpu.VMEM((_NODES_PER_WORKER, _FAN_IN), jnp.float32),
            pltpu.VMEM((_NODES_PER_WORKER, _FAN_IN), jnp.int32),
            pltpu.VMEM((_R_TOTAL,), jnp.float32),
            pltpu.VMEM((16,), jnp.float32),
        ],
        compiler_params=pltpu.CompilerParams(needs_layout_passes=False),
    )


def _recur_body(blk, b_tile, mt_ref, d8_ref, xT_ref, out_ref, acc_s, out_s, xf_s):
    n_blks = _N_NODES // blk
    xf_s[...] = xT_ref[...].astype(jnp.float32)
    for b in range(n_blks):
        bs = b * blk
        acc = jnp.dot(
            mt_ref[bs : bs + blk, : _IN_SIZE],
            xf_s[...],
            preferred_element_type=jnp.float32,
            precision=lax.Precision.HIGHEST,
        )
        if b > 0:
            acc = acc + jnp.dot(
                mt_ref[bs : bs + blk, _IN_SIZE : _IN_SIZE + bs],
                out_s[0:bs, :],
                preferred_element_type=jnp.float32,
                precision=lax.Precision.HIGHEST,
            )
        acc_s[...] = acc
        out_s[bs : bs + blk, :] = jnp.zeros((blk, b_tile), jnp.float32)

        def do_group(g, is_out=False):
            off16 = pl.multiple_of(g * 16, 8)
            base16 = pl.multiple_of(bs + g * 16, 8)
            m16 = mt_ref[pl.ds(base16, 16), _IN_SIZE + bs : _IN_SIZE + bs + blk]
            acc16 = acc_s[pl.ds(off16, 16), :] + jnp.dot(
                m16,
                out_s[bs : bs + blk, :],
                preferred_element_type=jnp.float32,
                precision=lax.Precision.HIGHEST,
            )
            d16 = d8_ref[pl.ds(base16, 16), :]
            rows_r, rows_f = [], []
            for k in range(16):
                v = acc16[k : k + 1, :]
                for j in range(k):
                    v = v + d16[k : k + 1, j : j + 1] * rows_r[j]
                t = jnp.tanh(v)
                rows_f.append(t)
                rows_r.append(t.astype(jnp.bfloat16).astype(jnp.float32))
            out_s[pl.ds(base16, 16), :] = jnp.concatenate(rows_r, axis=0)
            if is_out:
                out_ref[...] = jnp.concatenate(rows_f, axis=0)

        n_groups = blk // 16
        if b < n_blks - 1:
            lax.fori_loop(0, n_groups, lambda g, c: (do_group(g), c)[1], 0)
        else:
            lax.fori_loop(0, n_groups - 1, lambda g, c: (do_group(g), c)[1], 0)
            do_group(n_groups - 1, is_out=True)


def _make_recurrence(batch, b_tile, blk, interpret=False):
    return pl.pallas_call(
        functools.partial(_recur_body, blk, b_tile),
        grid=(batch // b_tile,),
        in_specs=[
            pl.BlockSpec((_N_NODES, _R_TOTAL), lambda i: (0, 0)),
            pl.BlockSpec((_N_NODES, 16), lambda i: (0, 0)),
            pl.BlockSpec((_IN_SIZE, b_tile), lambda i: (0, i)),
        ],
        out_specs=pl.BlockSpec((_OUT_SIZE, b_tile), lambda i: (0, i)),
        out_shape=jax.ShapeDtypeStruct((_OUT_SIZE, batch), jnp.float32),
        scratch_shapes=[
            pltpu.VMEM((blk, b_tile), jnp.float32),
            pltpu.VMEM((_N_NODES, b_tile), jnp.float32),
            pltpu.VMEM((_IN_SIZE, b_tile), jnp.float32),
        ],
        interpret=interpret,
    )


def kernel(x, weights, in_idxs):
    mt, d8 = _build_mt()(weights, in_idxs.astype(jnp.int32))
    xT = x.T.astype(jnp.bfloat16)
    return _make_recurrence(_BATCH, 1024, 128)(mt, d8, xT)


# speedup vs baseline: 67.5601x; 1.0978x over previous
import functools

import jax
import jax.numpy as jnp
from jax import lax
from jax.experimental import pallas as pl
from jax.experimental.pallas import tpu as pltpu
from jax.experimental.pallas import tpu_sc as plsc

_IN_SIZE = 1024
_N_NODES = 2048
_FAN_IN = 128
_OUT_SIZE = 16
_BATCH = 4096
_R_TOTAL = _IN_SIZE + _N_NODES

_N_WORKERS = 32
_NODES_PER_WORKER = _N_NODES // _N_WORKERS


def _build_mt_kernel(w_hbm, idx_hbm, mt_hbm, d8_hbm, w_v, idx_v, row_v, d8_v):
    wid = lax.axis_index("s") * 2 + lax.axis_index("c")
    base = wid * _NODES_PER_WORKER
    pltpu.sync_copy(w_hbm.at[pl.ds(base, _NODES_PER_WORKER)], w_v)
    pltpu.sync_copy(idx_hbm.at[pl.ds(base, _NODES_PER_WORKER)], idx_v)

    zeros16 = jnp.zeros((16,), jnp.float32)

    def zero_body(i, carry):
        row_v[pl.ds(i * 16, 16)] = zeros16
        return carry

    lax.fori_loop(0, _R_TOTAL // 16, zero_body, 0)
    d8_v[...] = zeros16

    def node_body(n, carry):
        node = base + n
        limit = _IN_SIZE + node
        group_ref_base = limit - lax.rem(node, 16)
        for v in range(_FAN_IN // 16):
            iv = idx_v[n, pl.ds(v * 16, 16)]
            wv = w_v[n, pl.ds(v * 16, 16)]
            iw = lax.bitcast_convert_type(wv, jnp.int32)
            lsb = jnp.bitwise_and(lax.shift_right_logical(iw, 16), 1)
            ir = jnp.bitwise_and(iw + 0x7FFF + lsb, jnp.int32(-65536))
            wr = lax.bitcast_convert_type(ir, jnp.float32)
            keep = iv < limit
            wv = jnp.where(keep, wr, 0.0)
            plsc.addupdate_scatter(row_v, [iv], wv)
            iv2 = iv - group_ref_base
            ing = jnp.logical_and(iv2 >= 0, keep)
            wv2 = jnp.where(ing, wr, 0.0)
            iv2c = jnp.clip(iv2, 0, 15)
            plsc.addupdate_scatter(d8_v, [iv2c], wv2)
        pltpu.sync_copy(row_v, mt_hbm.at[node])
        pltpu.sync_copy(d8_v, d8_hbm.at[node])
        for v in range(_FAN_IN // 16):
            iv = idx_v[n, pl.ds(v * 16, 16)]
            plsc.store_scatter(row_v, [iv], zeros16)
        d8_v[...] = zeros16
        return carry

    lax.fori_loop(0, _NODES_PER_WORKER, node_body, 0)


@functools.cache
def _build_mt():
    return pl.kernel(
        _build_mt_kernel,
        out_type=(
            jax.ShapeDtypeStruct((_N_NODES, _R_TOTAL), jnp.float32),
            jax.ShapeDtypeStruct((_N_NODES, 16), jnp.float32),
        ),
        mesh=plsc.VectorSubcoreMesh(core_axis_name="c", subcore_axis_name="s"),
        scratch_types=[
            pltpu.VMEM((_NODES_PER_WORKER, _FAN_IN), jnp.float32),
            pltpu.VMEM((_NODES_PER_WORKER, _FAN_IN), jnp.int32),
            pltpu.VMEM((_R_TOTAL,), jnp.float32),
            pltpu.VMEM((16,), jnp.float32),
        ],
        compiler_params=pltpu.CompilerParams(needs_layout_passes=False),
    )


def _recur_body(blk, b_tile, mt_ref, d8_ref, xT_ref, out_ref, acc_s, out_s, xf_s):
    n_blks = _N_NODES // blk
    xf_s[...] = xT_ref[...].astype(jnp.float32)
    for b in range(n_blks):
        bs = b * blk
        acc = jnp.dot(
            mt_ref[bs : bs + blk, : _IN_SIZE],
            xf_s[...],
            preferred_element_type=jnp.float32,
            precision=lax.Precision.HIGHEST,
        )
        if b > 0:
            acc = acc + jnp.dot(
                mt_ref[bs : bs + blk, _IN_SIZE : _IN_SIZE + bs],
                out_s[0:bs, :],
                preferred_element_type=jnp.float32,
                precision=lax.Precision.HIGHEST,
            )
        acc_s[...] = acc
        out_s[bs : bs + blk, :] = jnp.zeros((blk, b_tile), jnp.float32)

        def do_group(g, is_out=False):
            off16 = pl.multiple_of(g * 16, 8)
            base16 = pl.multiple_of(bs + g * 16, 8)
            m16 = mt_ref[pl.ds(base16, 16), _IN_SIZE + bs : _IN_SIZE + bs + blk]
            acc16 = acc_s[pl.ds(off16, 16), :] + jnp.dot(
                m16,
                out_s[bs : bs + blk, :],
                preferred_element_type=jnp.float32,
                precision=lax.Precision.HIGHEST,
            )
            d16 = d8_ref[pl.ds(base16, 16), :]
            rows_r, rows_f = [], []
            for k in range(16):
                v = acc16[k : k + 1, :]
                for j in range(k):
                    v = v + d16[k : k + 1, j : j + 1] * rows_r[j]
                t = jnp.tanh(v)
                rows_f.append(t)
                rows_r.append(t.astype(jnp.bfloat16).astype(jnp.float32))
            out_s[pl.ds(base16, 16), :] = jnp.concatenate(rows_r, axis=0)
            if is_out:
                out_ref[...] = jnp.concatenate(rows_f, axis=0)

        n_groups = blk // 16
        if b < n_blks - 1:
            lax.fori_loop(0, n_groups, lambda g, c: (do_group(g), c)[1], 0)
        else:
            lax.fori_loop(0, n_groups - 1, lambda g, c: (do_group(g), c)[1], 0)
            do_group(n_groups - 1, is_out=True)


def _make_recurrence(batch, b_tile, blk, interpret=False):
    return pl.pallas_call(
        functools.partial(_recur_body, blk, b_tile),
        grid=(batch // b_tile,),
        in_specs=[
            pl.BlockSpec((_N_NODES, _R_TOTAL), lambda i: (0, 0)),
            pl.BlockSpec((_N_NODES, 16), lambda i: (0, 0)),
            pl.BlockSpec((_IN_SIZE, b_tile), lambda i: (0, i)),
        ],
        out_specs=pl.BlockSpec((_OUT_SIZE, b_tile), lambda i: (0, i)),
        out_shape=jax.ShapeDtypeStruct((_OUT_SIZE, batch), jnp.float32),
        scratch_shapes=[
            pltpu.VMEM((blk, b_tile), jnp.float32),
            pltpu.VMEM((_N_NODES, b_tile), jnp.float32),
            pltpu.VMEM((_IN_SIZE, b_tile), jnp.float32),
        ],
        compiler_params=pltpu.CompilerParams(vmem_limit_bytes=64 * 1024 * 1024),
        interpret=interpret,
    )


def kernel(x, weights, in_idxs):
    mt, d8 = _build_mt()(weights, in_idxs.astype(jnp.int32))
    xT = x.T.astype(jnp.bfloat16)
    return _make_recurrence(_BATCH, 2048, 128)(mt, d8, xT)
